# baseline (speedup 1.0000x reference)
import functools

import jax
import jax.numpy as jnp
import numpy as np
from jax import lax
from jax.experimental import pallas as pl
from jax.experimental.pallas import tpu as pltpu
from jax.experimental.pallas import tpu_sc as plsc

B = 1024
L = 150
D = 25
V = 5001
H = 500
C = 2
DH = 12
DPAD = 16
NIDX = B * L

NC = 2
NS = 16
NW = NC * NS
B_PER_W = NIDX // NW
BPW = B // NW
GRP = 8
NG = BPW // GRP
WCR = L * DH + 8

_P = np.zeros((D, DPAD), np.float32)
for _j in range(DH):
    _P[2 * _j, _j] = 0.5
    _P[2 * _j + 1, _j] = 0.5


def _dyn_gather(x, idx):
    return lax.gather(
        x, idx[:, None],
        lax.GatherDimensionNumbers(offset_dims=(), collapsed_slice_dims=(0,),
                                   start_index_map=(0,)),
        (1,), mode=lax.GatherScatterMode.PROMISE_IN_BOUNDS)


def _precompute(emb, p_mat, w1, w2, b1r, b2r):
    def body(emb_ref, p_ref, w1_ref, w2_ref, b1_ref, b2_ref,
             pt_ref, wc_ref, bc_ref):
        pt_ref[...] = jnp.dot(emb_ref[...], p_ref[...],
                              preferred_element_type=jnp.float32)
        wc_ref[...] = jnp.dot(w1_ref[...], w2_ref[...],
                              preferred_element_type=jnp.float32)
        bcd = jnp.dot(b1_ref[...], w2_ref[...],
                      preferred_element_type=jnp.float32) + b2_ref[...]
        bc_ref[...] = jnp.concatenate([bcd] * (DPAD // C), axis=1)

    return pl.pallas_call(
        body,
        out_shape=[
            jax.ShapeDtypeStruct((V, DPAD), jnp.float32),
            jax.ShapeDtypeStruct((L * DH, C), jnp.float32),
            jax.ShapeDtypeStruct((1, DPAD), jnp.float32),
        ],
    )(emb, p_mat, w1, w2, b1r, b2r)


def _sc_fused(ptable, idx2, wc12, bct):
    mesh = plsc.VectorSubcoreMesh(core_axis_name="c", subcore_axis_name="s")

    @functools.partial(
        pl.kernel,
        mesh=mesh,
        out_type=jax.ShapeDtypeStruct((B * C,), jnp.float32),
        scratch_types=[
            pltpu.VMEM((V, DPAD), jnp.float32),
            pltpu.VMEM((B_PER_W,), jnp.int32),
            pltpu.VMEM((WCR, C), jnp.float32),
            pltpu.VMEM((L, C, DPAD), jnp.float32),
            pltpu.VMEM((1, DPAD), jnp.float32),
            pltpu.VMEM((BPW * C,), jnp.float32),
            pltpu.SemaphoreType.DMA,
        ],
        compiler_params=pltpu.CompilerParams(
            use_tc_tiling_on_sc=False, needs_layout_passes=False),
    )
    def k(pt_hbm, idx_hbm, wc_hbm, bc_hbm, out_hbm,
          pt_v, idx_v, wcr_v, wc_v, bc_v, out_v, sem):
        wid = lax.axis_index("s") * NC + lax.axis_index("c")
        table_cp = pltpu.make_async_copy(pt_hbm, pt_v, sem)
        table_cp.start()
        pltpu.sync_copy(idx_hbm.at[wid], idx_v)
        pltpu.sync_copy(wc_hbm, wcr_v.at[pl.ds(0, L * DH)])
        pltpu.sync_copy(bc_hbm, bc_v)

        lanes = lax.iota(jnp.int32, 16)
        lane_grp = lanes & (GRP - 1)

        def build_wc(l, carry):
            rows = l * DH + lanes
            for c in range(C):
                cols = jnp.full((16,), c, jnp.int32)
                vec = plsc.load_gather(wcr_v, [rows, cols])
                wc_v[l, c] = jnp.where(lanes < DH, vec, 0.0)
            return carry

        lax.fori_loop(0, L, build_wc, 0)
        bc_tiled = bc_v[0]

        table_cp.wait()

        zero16 = jnp.zeros((16,), jnp.float32)

        def group_body(g, carry):
            base_vec = (g * GRP + lane_grp) * L

            def l_body(l, accs):
                idx16 = plsc.load_gather(idx_v, [base_vec + l])
                w0 = wc_v[l, 0]
                w1 = wc_v[l, 1]
                new = []
                for kk in range(GRP):
                    spl = _dyn_gather(idx16, jnp.full((16,), kk, jnp.int32))
                    row = plsc.load_gather(pt_v, [spl, lanes])
                    new.append(accs[2 * kk] + row * w0)
                    new.append(accs[2 * kk + 1] + row * w1)
                return tuple(new)

            accs = lax.fori_loop(0, L, l_body, (zero16,) * (2 * GRP))
            out_vec = zero16
            for kk in range(GRP):
                s0 = jnp.sum(accs[2 * kk])
                s1 = jnp.sum(accs[2 * kk + 1])
                out_vec = jnp.where(lanes == 2 * kk, s0, out_vec)
                out_vec = jnp.where(lanes == 2 * kk + 1, s1, out_vec)
            out_v[pl.ds(g * 16, 16)] = out_vec + bc_tiled
            return carry

        lax.fori_loop(0, NG, group_body, 0)
        pltpu.sync_copy(out_v, out_hbm.at[pl.ds(wid * (BPW * C), BPW * C)])

    return k(ptable, idx2, wc12, bct)


def kernel(sents, _, emb_table, W1, b1, W2, b2):
    ptable, wc12, bct = _precompute(
        emb_table, jnp.asarray(_P), W1, W2,
        b1.reshape(1, H), b2.reshape(1, C))
    idx2 = sents.astype(jnp.int32).reshape(NW, B_PER_W)
    out = _sc_fused(ptable, idx2, wc12, bct)
    return out.reshape(B, C)
